# Initial kernel scaffold; baseline (speedup 1.0000x reference)
import jax
import jax.numpy as jnp
from jax.experimental import pallas as pl


def kernel(graph, x, W, b):
    raise NotImplementedError("write your pallas kernel here")


# speedup vs baseline: 8.2091x; 8.2091x over previous
import functools

import jax
import jax.numpy as jnp
from jax import lax
from jax.experimental import pallas as pl
from jax.experimental.pallas import tpu as pltpu
from jax.experimental.pallas import tpu_sc as plsc

N = 10000
E = 320000
D_IN = 128
D_OUT = 128
NC = 2
NS = 16
NW = NC * NS
B = 128
K = 79
EPT = K * B
EP = NW * EPT
DUMP = 240
SP_ROWS = N + DUMP


def _matmul_body(x_ref, w_ref, b_ref, o_ref):
    o_ref[...] = lax.dot_general(
        x_ref[...], w_ref[...], (((1,), (1,)), ((), ())),
        preferred_element_type=jnp.float32) + b_ref[...]


def _linear(x, W, b):
    m_blk = 1000
    return pl.pallas_call(
        _matmul_body,
        grid=(N // m_blk,),
        in_specs=[
            pl.BlockSpec((m_blk, D_IN), lambda i: (i, 0)),
            pl.BlockSpec((D_OUT, D_IN), lambda i: (0, 0)),
            pl.BlockSpec((1, D_OUT), lambda i: (0, 0)),
        ],
        out_specs=pl.BlockSpec((m_blk, D_OUT), lambda i: (i, 0)),
        out_shape=jax.ShapeDtypeStruct((N, D_OUT), jnp.float32),
    )(x, W, b.reshape(1, D_OUT))


def _add_body(a_ref, b_ref, o_ref):
    o_ref[...] = a_ref[...] + b_ref[...]


def _sum_partials(p0, p1):
    m_blk = 1000
    return pl.pallas_call(
        _add_body,
        grid=(N // m_blk,),
        in_specs=[
            pl.BlockSpec((m_blk, D_OUT), lambda i: (i, 0)),
            pl.BlockSpec((m_blk, D_OUT), lambda i: (i, 0)),
        ],
        out_specs=pl.BlockSpec((m_blk, D_OUT), lambda i: (i, 0)),
        out_shape=jax.ShapeDtypeStruct((N, D_OUT), jnp.float32),
    )(p0, p1)


def _make_scatter():
    mesh = plsc.VectorSubcoreMesh(
        core_axis_name="c", subcore_axis_name="s",
        num_cores=NC, num_subcores=NS)

    @functools.partial(
        pl.kernel,
        out_type=jax.ShapeDtypeStruct((NC, SP_ROWS, D_OUT), jnp.float32),
        mesh=mesh,
        scratch_types=[
            pltpu.VMEM((K, B), jnp.int32),
            pltpu.VMEM((K, B), jnp.int32),
            pltpu.VMEM((B, D_OUT), jnp.float32),
            pltpu.VMEM_SHARED((SP_ROWS, D_OUT), jnp.float32),
            pltpu.SemaphoreType.DMA,
        ],
    )
    def scatter(h, srcs, dsts, zeros, out, src_v, dst_v, rows_v, acc, sem):
        c = lax.axis_index("c")
        s = lax.axis_index("s")
        w = c * NS + s

        zr = SP_ROWS // NS
        pltpu.sync_copy(zeros.at[pl.ds(s * zr, zr)], acc.at[pl.ds(s * zr, zr)])
        pltpu.sync_copy(srcs.at[w], src_v)
        pltpu.sync_copy(dsts.at[w], dst_v)
        plsc.subcore_barrier()

        def chunk(j, carry):
            pltpu.async_copy(h.at[src_v.at[j]], rows_v, sem).wait()
            pltpu.sync_copy(rows_v, acc.at[dst_v.at[j]], add=True)
            return carry

        lax.fori_loop(0, K, chunk, 0)
        plsc.subcore_barrier()

        pltpu.sync_copy(acc.at[pl.ds(s * zr, zr)],
                        out.at[c, pl.ds(s * zr, zr)])

    return scatter


_scatter = _make_scatter()


def kernel(graph, x, W, b):
    h = _linear(x, W, b)
    src = graph[0]
    dst = graph[1]
    npad = EP - E
    pad_src = (jnp.arange(npad, dtype=jnp.int32) * 37) % N
    pad_dst = N + (jnp.arange(npad, dtype=jnp.int32) % DUMP)
    srcs = jnp.concatenate([src, pad_src]).reshape(NW, K, B)
    dsts = jnp.concatenate([dst, pad_dst]).reshape(NW, K, B)
    zeros = jnp.zeros((SP_ROWS, D_OUT), jnp.float32)
    parts = _scatter(h, srcs, dsts, zeros)
    return _sum_partials(parts[0, :N], parts[1, :N])


# speedup vs baseline: 9.4776x; 1.1545x over previous
import functools

import jax
import jax.numpy as jnp
from jax import lax
from jax.experimental import pallas as pl
from jax.experimental.pallas import tpu as pltpu
from jax.experimental.pallas import tpu_sc as plsc

N = 10000
E = 320000
D_IN = 128
D_OUT = 128
NC = 2
NS = 16
NW = NC * NS
B = 128
K = 80
NB = 2
G = K // NB
EPT = K * B
EP = NW * EPT
DUMP = 240
SP_ROWS = N + DUMP


def _matmul_body(x_ref, w_ref, b_ref, o_ref):
    o_ref[...] = lax.dot_general(
        x_ref[...], w_ref[...], (((1,), (1,)), ((), ())),
        preferred_element_type=jnp.float32) + b_ref[...]


def _linear(x, W, b):
    m_blk = 1000
    return pl.pallas_call(
        _matmul_body,
        grid=(N // m_blk,),
        in_specs=[
            pl.BlockSpec((m_blk, D_IN), lambda i: (i, 0)),
            pl.BlockSpec((D_OUT, D_IN), lambda i: (0, 0)),
            pl.BlockSpec((1, D_OUT), lambda i: (0, 0)),
        ],
        out_specs=pl.BlockSpec((m_blk, D_OUT), lambda i: (i, 0)),
        out_shape=jax.ShapeDtypeStruct((N, D_OUT), jnp.float32),
    )(x, W, b.reshape(1, D_OUT))


def _add_body(a_ref, b_ref, o_ref):
    o_ref[...] = a_ref[...] + b_ref[...]


def _sum_partials(p0, p1):
    m_blk = 1000
    return pl.pallas_call(
        _add_body,
        grid=(N // m_blk,),
        in_specs=[
            pl.BlockSpec((m_blk, D_OUT), lambda i: (i, 0)),
            pl.BlockSpec((m_blk, D_OUT), lambda i: (i, 0)),
        ],
        out_specs=pl.BlockSpec((m_blk, D_OUT), lambda i: (i, 0)),
        out_shape=jax.ShapeDtypeStruct((N, D_OUT), jnp.float32),
    )(p0, p1)


def _make_scatter():
    mesh = plsc.VectorSubcoreMesh(
        core_axis_name="c", subcore_axis_name="s",
        num_cores=NC, num_subcores=NS)

    @functools.partial(
        pl.kernel,
        out_type=jax.ShapeDtypeStruct((NC, SP_ROWS, D_OUT), jnp.float32),
        mesh=mesh,
        scratch_types=[
            pltpu.VMEM((K, B), jnp.int32),
            pltpu.VMEM((NB, B), jnp.int32),
            pltpu.VMEM((NB, B, D_OUT), jnp.float32),
            pltpu.VMEM_SHARED((SP_ROWS, D_OUT), jnp.float32),
            pltpu.SemaphoreType.DMA((NB,)),
            pltpu.SemaphoreType.DMA((NB,)),
            pltpu.SemaphoreType.DMA((NB,)),
        ],
    )
    def scatter(h, srcs, dsts, zeros, out, src_v, dst_v, rows_v, acc,
                g_sem, d_sem, s_sem):
        c = lax.axis_index("c")
        s = lax.axis_index("s")
        w = c * NS + s

        zr = SP_ROWS // NS
        pltpu.sync_copy(zeros.at[pl.ds(s * zr, zr)], acc.at[pl.ds(s * zr, zr)])
        pltpu.sync_copy(srcs.at[w], src_v)
        plsc.subcore_barrier()

        def gather_start(j, b):
            pltpu.async_copy(h.at[src_v.at[j]], rows_v.at[b], g_sem.at[b])
            pltpu.async_copy(dsts.at[w, j], dst_v.at[b], d_sem.at[b])

        def gather_wait(j, b):
            pltpu.make_async_copy(h.at[src_v.at[j]], rows_v.at[b],
                                  g_sem.at[b]).wait()
            pltpu.make_async_copy(dsts.at[w, j], dst_v.at[b],
                                  d_sem.at[b]).wait()

        def scat_start(j, b):
            pltpu.async_copy(rows_v.at[b], acc.at[dst_v.at[b]], s_sem.at[b],
                             add=True)

        def scat_wait(j, b):
            pltpu.make_async_copy(rows_v.at[b], acc.at[dst_v.at[b]],
                                  s_sem.at[b]).wait()

        for b in range(NB):
            gather_start(b, b)

        def group(g, carry):
            for b in range(NB):
                j = g * NB + b
                gather_wait(j, b)
                scat_start(j, b)
            for b in range(NB):
                j = g * NB + b
                scat_wait(j, b)
                gather_start(j + NB, b)
            return carry

        lax.fori_loop(0, G - 1, group, 0)

        for b in range(NB):
            j = (G - 1) * NB + b
            gather_wait(j, b)
            scat_start(j, b)
        for b in range(NB):
            j = (G - 1) * NB + b
            scat_wait(j, b)
        plsc.subcore_barrier()

        pltpu.sync_copy(acc.at[pl.ds(s * zr, zr)],
                        out.at[c, pl.ds(s * zr, zr)])

    return scatter


_scatter = _make_scatter()


def kernel(graph, x, W, b):
    h = _linear(x, W, b)
    src = graph[0]
    dst = graph[1]
    npad = EP - E
    pad_src = (jnp.arange(npad, dtype=jnp.int32) * 37) % N
    pad_dst = N + (jnp.arange(npad, dtype=jnp.int32) % DUMP)
    srcs = jnp.concatenate([src, pad_src]).reshape(NW, K, B)
    dsts = jnp.concatenate([dst, pad_dst]).reshape(NW, K, B)
    zeros = jnp.zeros((SP_ROWS, D_OUT), jnp.float32)
    parts = _scatter(h, srcs, dsts, zeros)
    return _sum_partials(parts[0, :N], parts[1, :N])


# speedup vs baseline: 9.8494x; 1.0392x over previous
import functools

import jax
import jax.numpy as jnp
from jax import lax
from jax.experimental import pallas as pl
from jax.experimental.pallas import tpu as pltpu
from jax.experimental.pallas import tpu_sc as plsc

N = 10000
E = 320000
D_IN = 128
D_OUT = 128
NC = 2
NS = 16
NW = NC * NS
EPT = E // NW
B = 128
K = EPT // B
TB = EPT - K * B
NB = 2
G = K // NB


def _matmul_body(x_ref, w_ref, b_ref, o_ref):
    o_ref[...] = lax.dot_general(
        x_ref[...], w_ref[...], (((1,), (1,)), ((), ())),
        preferred_element_type=jnp.float32) + b_ref[...]


def _linear(x, W, b):
    m_blk = 1000
    return pl.pallas_call(
        _matmul_body,
        grid=(N // m_blk,),
        in_specs=[
            pl.BlockSpec((m_blk, D_IN), lambda i: (i, 0)),
            pl.BlockSpec((D_OUT, D_IN), lambda i: (0, 0)),
            pl.BlockSpec((1, D_OUT), lambda i: (0, 0)),
        ],
        out_specs=pl.BlockSpec((m_blk, D_OUT), lambda i: (i, 0)),
        out_shape=jax.ShapeDtypeStruct((N, D_OUT), jnp.float32),
    )(x, W, b.reshape(1, D_OUT))


def _add_body(a_ref, b_ref, o_ref):
    o_ref[...] = a_ref[0] + b_ref[0]


def _sum_partials(parts):
    m_blk = 1000
    return pl.pallas_call(
        _add_body,
        grid=(N // m_blk,),
        in_specs=[
            pl.BlockSpec((1, m_blk, D_OUT), lambda i: (0, i, 0)),
            pl.BlockSpec((1, m_blk, D_OUT), lambda i: (1, i, 0)),
        ],
        out_specs=pl.BlockSpec((m_blk, D_OUT), lambda i: (i, 0)),
        out_shape=jax.ShapeDtypeStruct((N, D_OUT), jnp.float32),
    )(parts, parts)


def _make_scatter():
    mesh = plsc.VectorSubcoreMesh(
        core_axis_name="c", subcore_axis_name="s",
        num_cores=NC, num_subcores=NS)

    @functools.partial(
        pl.kernel,
        out_type=jax.ShapeDtypeStruct((NC, N, D_OUT), jnp.float32),
        mesh=mesh,
        scratch_types=[
            pltpu.VMEM((EPT,), jnp.int32),
            pltpu.VMEM((NB, B), jnp.int32),
            pltpu.VMEM((NB, B, D_OUT), jnp.float32),
            pltpu.VMEM((TB,), jnp.int32),
            pltpu.VMEM((TB, D_OUT), jnp.float32),
            pltpu.VMEM_SHARED((N, D_OUT), jnp.float32),
            pltpu.SemaphoreType.DMA((NB,)),
            pltpu.SemaphoreType.DMA((NB,)),
            pltpu.SemaphoreType.DMA((NB,)),
            pltpu.SemaphoreType.DMA,
        ],
    )
    def scatter(h, srcs, dsts, zeros, out, src_v, dst_v, rows_v, tdst_v,
                trows_v, acc, g_sem, d_sem, s_sem, t_sem):
        c = lax.axis_index("c")
        s = lax.axis_index("s")
        w = c * NS + s

        @pl.when(s < NS - 1)
        def _():
            pltpu.sync_copy(zeros.at[pl.ds(s * 640, 640)],
                            acc.at[pl.ds(s * 640, 640)])

        @pl.when(s == NS - 1)
        def _():
            pltpu.sync_copy(zeros.at[pl.ds(9600, 400)],
                            acc.at[pl.ds(9600, 400)])

        pltpu.sync_copy(srcs.at[w], src_v)
        plsc.subcore_barrier()

        def gather_start(j, b):
            pltpu.async_copy(h.at[src_v.at[pl.ds(j * B, B)]], rows_v.at[b],
                             g_sem.at[b])
            pltpu.async_copy(dsts.at[w, pl.ds(j * B, B)], dst_v.at[b],
                             d_sem.at[b])

        def gather_wait(j, b):
            pltpu.make_async_copy(h.at[src_v.at[pl.ds(j * B, B)]],
                                  rows_v.at[b], g_sem.at[b]).wait()
            pltpu.make_async_copy(dsts.at[w, pl.ds(j * B, B)], dst_v.at[b],
                                  d_sem.at[b]).wait()

        def scat_start(j, b):
            pltpu.async_copy(rows_v.at[b], acc.at[dst_v.at[b]], s_sem.at[b],
                             add=True)

        def scat_wait(j, b):
            pltpu.make_async_copy(rows_v.at[b], acc.at[dst_v.at[b]],
                                  s_sem.at[b]).wait()

        pltpu.async_copy(h.at[src_v.at[pl.ds(K * B, TB)]], trows_v, t_sem)
        pltpu.async_copy(dsts.at[w, pl.ds(K * B, TB)], tdst_v, t_sem)

        for b in range(NB):
            gather_start(b, b)

        def group(g, carry):
            for b in range(NB):
                j = g * NB + b
                gather_wait(j, b)
                scat_start(j, b)
            for b in range(NB):
                j = g * NB + b
                scat_wait(j, b)
                gather_start(j + NB, b)
            return carry

        lax.fori_loop(0, G - 1, group, 0)

        for b in range(NB):
            j = (G - 1) * NB + b
            gather_wait(j, b)
            scat_start(j, b)

        pltpu.make_async_copy(h.at[src_v.at[pl.ds(K * B, TB)]], trows_v,
                              t_sem).wait()
        pltpu.make_async_copy(dsts.at[w, pl.ds(K * B, TB)], tdst_v,
                              t_sem).wait()
        pltpu.sync_copy(trows_v, acc.at[tdst_v], add=True)

        for b in range(NB):
            j = (G - 1) * NB + b
            scat_wait(j, b)
        plsc.subcore_barrier()

        @pl.when(s < NS - 1)
        def _():
            pltpu.sync_copy(acc.at[pl.ds(s * 640, 640)],
                            out.at[c, pl.ds(s * 640, 640)])

        @pl.when(s == NS - 1)
        def _():
            pltpu.sync_copy(acc.at[pl.ds(9600, 400)],
                            out.at[c, pl.ds(9600, 400)])

    return scatter


_scatter = _make_scatter()


def kernel(graph, x, W, b):
    h = _linear(x, W, b)
    srcs = graph[0].reshape(NW, EPT)
    dsts = graph[1].reshape(NW, EPT)
    zeros = jnp.zeros((N, D_OUT), jnp.float32)
    parts = _scatter(h, srcs, dsts, zeros)
    return _sum_partials(parts)


# speedup vs baseline: 10.9383x; 1.1106x over previous
import functools

import jax
import jax.numpy as jnp
from jax import lax
from jax.experimental import pallas as pl
from jax.experimental.pallas import tpu as pltpu
from jax.experimental.pallas import tpu_sc as plsc

N = 10000
E = 320000
D_IN = 128
D_OUT = 128
NC = 2
NS = 16
NW = NC * NS
EPT = E // NW
B = 128
K = EPT // B
TB = EPT - K * B
G = K // 2
ZR = 640


def _matmul_body(x_ref, w_ref, b_ref, o_ref):
    o_ref[...] = lax.dot_general(
        x_ref[...], w_ref[...], (((1,), (1,)), ((), ())),
        preferred_element_type=jnp.float32) + b_ref[...]


def _linear(x, W, b):
    m_blk = 2000
    return pl.pallas_call(
        _matmul_body,
        grid=(N // m_blk,),
        in_specs=[
            pl.BlockSpec((m_blk, D_IN), lambda i: (i, 0)),
            pl.BlockSpec((D_OUT, D_IN), lambda i: (0, 0)),
            pl.BlockSpec((1, D_OUT), lambda i: (0, 0)),
        ],
        out_specs=pl.BlockSpec((m_blk, D_OUT), lambda i: (i, 0)),
        out_shape=jax.ShapeDtypeStruct((N, D_OUT), jnp.float32),
    )(x, W, b.reshape(1, D_OUT))


def _add_body(a_ref, b_ref, o_ref):
    o_ref[...] = a_ref[0] + b_ref[0]


def _sum_partials(parts):
    m_blk = 2000
    return pl.pallas_call(
        _add_body,
        grid=(N // m_blk,),
        in_specs=[
            pl.BlockSpec((1, m_blk, D_OUT), lambda i: (0, i, 0)),
            pl.BlockSpec((1, m_blk, D_OUT), lambda i: (1, i, 0)),
        ],
        out_specs=pl.BlockSpec((m_blk, D_OUT), lambda i: (i, 0)),
        out_shape=jax.ShapeDtypeStruct((N, D_OUT), jnp.float32),
    )(parts, parts)


def _make_scatter():
    mesh = plsc.VectorSubcoreMesh(
        core_axis_name="c", subcore_axis_name="s",
        num_cores=NC, num_subcores=NS)

    @functools.partial(
        pl.kernel,
        out_type=jax.ShapeDtypeStruct((NC, N, D_OUT), jnp.float32),
        mesh=mesh,
        scratch_types=[
            pltpu.VMEM((EPT,), jnp.int32),
            pltpu.VMEM((2, B), jnp.int32),
            pltpu.VMEM((2, B, D_OUT), jnp.float32),
            pltpu.VMEM((TB,), jnp.int32),
            pltpu.VMEM((TB, D_OUT), jnp.float32),
            pltpu.VMEM_SHARED((N, D_OUT), jnp.float32),
            pltpu.SemaphoreType.DMA((2,)),
            pltpu.SemaphoreType.DMA((2,)),
            pltpu.SemaphoreType.DMA((2,)),
            pltpu.SemaphoreType.DMA,
        ],
    )
    def scatter(h, srcs, dsts, zeros, out, src_v, dst_v, rows_v, tdst_v,
                trows_v, acc, g_sem, d_sem, s_sem, t_sem):
        c = lax.axis_index("c")
        s = lax.axis_index("s")
        w = c * NS + s
        e0 = w * EPT

        @pl.when(s < NS - 1)
        def _():
            pltpu.sync_copy(zeros.at[pl.ds(0, ZR)],
                            acc.at[pl.ds(s * ZR, ZR)])

        @pl.when(s == NS - 1)
        def _():
            pltpu.sync_copy(zeros.at[pl.ds(0, N - (NS - 1) * ZR)],
                            acc.at[pl.ds((NS - 1) * ZR, N - (NS - 1) * ZR)])

        pltpu.sync_copy(srcs.at[w], src_v)
        plsc.subcore_barrier()

        def gather_start(j, b):
            pltpu.async_copy(h.at[src_v.at[pl.ds(j * B, B)]], rows_v.at[b],
                             g_sem.at[b])
            pltpu.async_copy(dsts.at[w, pl.ds(j * B, B)], dst_v.at[b],
                             d_sem.at[b])

        def gather_wait(j, b):
            pltpu.make_async_copy(h.at[src_v.at[pl.ds(j * B, B)]],
                                  rows_v.at[b], g_sem.at[b]).wait()
            pltpu.make_async_copy(dsts.at[w, pl.ds(j * B, B)],
                                  dst_v.at[b], d_sem.at[b]).wait()

        def scat_start(j, b):
            pltpu.async_copy(rows_v.at[b], acc.at[dst_v.at[b]], s_sem.at[b],
                             add=True)

        def scat_wait(j, b):
            pltpu.make_async_copy(rows_v.at[b], acc.at[dst_v.at[b]],
                                  s_sem.at[b]).wait()

        pltpu.async_copy(h.at[src_v.at[pl.ds(K * B, TB)]], trows_v, t_sem)
        pltpu.async_copy(dsts.at[w, pl.ds(K * B, TB)], tdst_v, t_sem)

        gather_start(0, 0)
        gather_wait(0, 0)
        scat_start(0, 0)
        gather_start(1, 1)
        gather_wait(1, 1)
        scat_start(1, 1)
        scat_wait(0, 0)
        gather_start(2, 0)

        def pair(g, carry):
            j0 = 2 * g
            j1 = j0 + 1
            gather_wait(j0, 0)
            scat_start(j0, 0)
            scat_wait(j1 - 2, 1)
            gather_start(j1, 1)
            gather_wait(j1, 1)
            scat_start(j1, 1)
            scat_wait(j0, 0)
            gather_start(j0 + 2, 0)
            return carry

        lax.fori_loop(1, G - 1, pair, 0)

        j0 = 2 * (G - 1)
        j1 = j0 + 1
        gather_wait(j0, 0)
        scat_start(j0, 0)
        scat_wait(j1 - 2, 1)
        gather_start(j1, 1)
        gather_wait(j1, 1)
        scat_start(j1, 1)
        scat_wait(j0, 0)

        pltpu.make_async_copy(h.at[src_v.at[pl.ds(K * B, TB)]], trows_v,
                              t_sem).wait()
        pltpu.make_async_copy(dsts.at[w, pl.ds(K * B, TB)], tdst_v,
                              t_sem).wait()
        pltpu.sync_copy(trows_v, acc.at[tdst_v], add=True)

        scat_wait(j1, 1)
        plsc.subcore_barrier()

        @pl.when(s < NS - 1)
        def _():
            pltpu.sync_copy(acc.at[pl.ds(s * ZR, ZR)],
                            out.at[c, pl.ds(s * ZR, ZR)])

        @pl.when(s == NS - 1)
        def _():
            pltpu.sync_copy(acc.at[pl.ds((NS - 1) * ZR, N - (NS - 1) * ZR)],
                            out.at[c, pl.ds((NS - 1) * ZR, N - (NS - 1) * ZR)])

    return scatter


_scatter = _make_scatter()


def kernel(graph, x, W, b):
    h = _linear(x, W, b)
    srcs = graph[0].reshape(NW, EPT)
    dsts = graph[1].reshape(NW, EPT)
    zeros = jnp.zeros((ZR, D_OUT), jnp.float32)
    parts = _scatter(h, srcs, dsts, zeros)
    return _sum_partials(parts)
